# baseline (speedup 1.0000x reference)
import functools

import jax
import jax.numpy as jnp
from jax.experimental import pallas as pl
from jax.experimental.pallas import tpu as pltpu

F32 = jnp.float32
_PREC = jax.lax.Precision.DEFAULT
_EXACT = jax.lax.Precision.HIGHEST

NB_CODE = 1024
NQ = 2


def _dot(a, b, prec=_PREC):
    return jax.lax.dot_general(a, b, (((1,), (0,)), ((), ())),
                               precision=prec, preferred_element_type=F32)


def _round8(n):
    return (n + 7) // 8 * 8


def _c1_body(x_ref, w_ref, b_ref, o_ref, *, K, dil, T_out, pre_relu, post_relu):
    x = x_ref[0]
    if pre_relu:
        x = jnp.maximum(x, 0.0)
    acc = _dot(x[0:T_out, :], w_ref[0])
    for k in range(1, K):
        acc = acc + _dot(x[k * dil:k * dil + T_out, :], w_ref[k])
    acc = acc + b_ref[:]
    if post_relu:
        acc = jnp.maximum(acc, 0.0)
    o_ref[0] = acc


def _c1_res_body(x_ref, w_ref, b_ref, r_ref, o_ref, *, K, dil, T_out,
                 pre_relu, post_relu):
    x = x_ref[0]
    if pre_relu:
        x = jnp.maximum(x, 0.0)
    acc = _dot(x[0:T_out, :], w_ref[0])
    for k in range(1, K):
        acc = acc + _dot(x[k * dil:k * dil + T_out, :], w_ref[k])
    acc = acc + b_ref[:]
    if post_relu:
        acc = jnp.maximum(acc, 0.0)
    o_ref[0] = acc + r_ref[0]


def conv1(x, w, b, *, padding=0, dilation=1, pre_relu=False, post_relu=False,
          residual=None):
    N, T, Ci = x.shape
    O, I, K = w.shape
    T_out = T + 2 * padding - dilation * (K - 1)
    wt = jnp.transpose(w, (2, 1, 0))
    b2 = b.reshape(1, O)
    Tp = _round8(T + 2 * padding)
    xp = jnp.pad(x, ((0, 0), (padding, Tp - T - padding), (0, 0)))
    in_specs = [
        pl.BlockSpec((1, Tp, Ci), lambda n: (n, 0, 0)),
        pl.BlockSpec((K, I, O), lambda n: (0, 0, 0)),
        pl.BlockSpec((1, O), lambda n: (0, 0)),
    ]
    args = [xp, wt, b2]
    if residual is not None:
        in_specs.append(pl.BlockSpec((1, T_out, O), lambda n: (n, 0, 0)))
        args.append(residual)
        body = functools.partial(_c1_res_body, K=K, dil=dilation, T_out=T_out,
                                 pre_relu=pre_relu, post_relu=post_relu)
    else:
        body = functools.partial(_c1_body, K=K, dil=dilation, T_out=T_out,
                                 pre_relu=pre_relu, post_relu=post_relu)
    return pl.pallas_call(
        body,
        grid=(N,),
        in_specs=in_specs,
        out_specs=pl.BlockSpec((1, T_out, O), lambda n: (n, 0, 0)),
        out_shape=jax.ShapeDtypeStruct((N, T_out, O), F32),
    )(*args)


def _c2_body(e_ref, o_ref_in, w_ref, b_ref, out_ref, *, T_out):
    e = e_ref[0]
    o = o_ref_in[0]
    acc = _dot(e[0:T_out, :], w_ref[0])
    acc = acc + _dot(o[0:T_out, :], w_ref[1])
    acc = acc + _dot(e[1:1 + T_out, :], w_ref[2])
    acc = acc + _dot(o[1:1 + T_out, :], w_ref[3])
    out_ref[0] = acc + b_ref[:]


def conv_s2(x, w, b):
    N, T, Ci = x.shape
    O, I, K = w.shape
    assert K == 4
    T_out = T // 2
    wt = jnp.transpose(w, (2, 1, 0))
    b2 = b.reshape(1, O)
    Th = T_out + 2
    Th8 = _round8(Th)
    xp = jnp.pad(x, ((0, 0), (1, 2 * Th8 - T - 1), (0, 0)))
    xph = xp.reshape(N, Th8, 2, Ci)
    ev = xph[:, :, 0, :]
    od = xph[:, :, 1, :]
    return pl.pallas_call(
        functools.partial(_c2_body, T_out=T_out),
        grid=(N,),
        in_specs=[
            pl.BlockSpec((1, Th8, Ci), lambda n: (n, 0, 0)),
            pl.BlockSpec((1, Th8, Ci), lambda n: (n, 0, 0)),
            pl.BlockSpec((K, I, O), lambda n: (0, 0, 0)),
            pl.BlockSpec((1, O), lambda n: (0, 0)),
        ],
        out_specs=pl.BlockSpec((1, T_out, O), lambda n: (n, 0, 0)),
        out_shape=jax.ShapeDtypeStruct((N, T_out, O), F32),
    )(ev, od, wt, b2)


def _vq_body(xf_ref, cb_ref, q_ref, idx_ref, stats_ref):
    xf = xf_ref[:]
    M, C = xf.shape
    residual = xf
    qout = jnp.zeros_like(xf)
    lane = jax.lax.broadcasted_iota(jnp.int32, (M, NB_CODE), 1)
    for q in range(NQ):
        cb = cb_ref[q]
        rn = jnp.sum(residual * residual, axis=1, keepdims=True)
        cn = jnp.sum(cb * cb, axis=1, keepdims=True)
        cross = jax.lax.dot_general(residual, cb, (((1,), (1,)), ((), ())),
                                    precision=_PREC,
                                    preferred_element_type=F32)
        d = rn - 2.0 * cross + jnp.transpose(cn)
        dmin = jnp.min(d, axis=1, keepdims=True)
        idx = jnp.min(jnp.where(d == dmin, lane, NB_CODE), axis=1,
                      keepdims=True)
        onehot = (lane == idx).astype(F32)
        qv = _dot(onehot, cb, _EXACT)
        diff = residual - qv
        closs = jnp.sum(diff * diff, keepdims=True).reshape(1, 1) / (M * C)
        qout = qout + (residual + (qv - residual))
        residual = diff
        pr = jnp.sum(onehot, axis=0, keepdims=True) / M
        ent = -jnp.sum(pr * jnp.log(pr + 1e-10), axis=1,
                       keepdims=True)
        idx_ref[q] = idx
        stats_ref[q:q + 1, 0:1] = closs
        stats_ref[q:q + 1, 1:2] = jnp.exp(ent)
    q_ref[:] = qout


def residual_vq_pallas(xf, codebooks):
    M, C = xf.shape
    qout, idx, stats = pl.pallas_call(
        _vq_body,
        in_specs=[
            pl.BlockSpec(memory_space=pltpu.VMEM),
            pl.BlockSpec(memory_space=pltpu.VMEM),
        ],
        out_specs=[
            pl.BlockSpec(memory_space=pltpu.VMEM),
            pl.BlockSpec(memory_space=pltpu.VMEM),
            pl.BlockSpec(memory_space=pltpu.VMEM),
        ],
        out_shape=[
            jax.ShapeDtypeStruct((M, C), F32),
            jax.ShapeDtypeStruct((NQ, M, 1), jnp.int32),
            jax.ShapeDtypeStruct((NQ, 2), F32),
        ],
    )(xf, codebooks)
    return qout, idx, stats


DOWN_T = 3
STRIDE_T = 2
DEPTH = 3
RATE = 3
DILS = [RATE ** d for d in range(DEPTH)][::-1]


def _resblock(h, rb, dil):
    y = conv1(h, rb['w1'], rb['b1'], padding=dil, dilation=dil,
              pre_relu=True, post_relu=True)
    return conv1(y, rb['w2'], rb['b2'], residual=h)


def _encoder(x, p):
    h = conv1(x, p['in_w'], p['in_b'], padding=1, post_relu=True)
    for blk in p['downs']:
        h = conv_s2(h, blk['dw'], blk['db'])
        for rb, dil in zip(blk['res'], DILS):
            h = _resblock(h, rb, dil)
    return conv1(h, p['out_w'], p['out_b'], padding=1)


def _decoder(z, p):
    h = conv1(z, p['in_w'], p['in_b'], padding=1, post_relu=True)
    for blk in p['ups']:
        for rb, dil in zip(blk['res'], DILS):
            h = _resblock(h, rb, dil)
        h = jnp.repeat(h, 2, axis=1)
        h = conv1(h, blk['uw'], blk['ub'], padding=1)
    h = conv1(h, p['mid_w'], p['mid_b'], padding=1, post_relu=True)
    return conv1(h, p['fin_w'], p['fin_b'], padding=1)


def kernel(x, params):
    x = x.astype(F32)
    N = x.shape[0]
    x_enc = _encoder(x, params['encoder'])
    Te, C = x_enc.shape[1], x_enc.shape[2]
    xf = x_enc.reshape(N * Te, C)
    qout, idx, stats = residual_vq_pallas(xf, params['codebooks'])
    x_q = qout.reshape(N, Te, C)
    x_out = _decoder(x_q, params['decoder'])
    code_idx = jnp.transpose(idx.reshape(NQ, N, Te), (1, 2, 0))
    commit = jnp.sum(stats[:, 0])
    perp = jnp.mean(stats[:, 1])
    return (x_out, code_idx, commit, perp)
